# baseline (speedup 1.0000x reference)
import functools

import jax
import jax.numpy as jnp
from jax import lax
from jax.experimental import pallas as pl
from jax.experimental.pallas import tpu as pltpu
from jax.experimental.pallas import tpu_sc as plsc

_B = 4096
_T = 50
_D = 64
_NC = 2
_NS = 16
_NW = _NC * _NS
_BPW = _B // _NW
_CB = 8
_NCHUNK = _BPW // _CB
_ROWS = _CB * _T

_mesh = plsc.VectorSubcoreMesh(core_axis_name="c", subcore_axis_name="s",
                               num_cores=_NC)


@functools.partial(
    pl.kernel,
    out_type=[
        jax.ShapeDtypeStruct((_B, _D), jnp.float32),
        jax.ShapeDtypeStruct((_B, 64), jnp.float32),
        jax.ShapeDtypeStruct((_B, 32), jnp.float32),
        jax.ShapeDtypeStruct((_B, _D), jnp.float32),
        jax.ShapeDtypeStruct((_B,), jnp.float32),
    ],
    mesh=_mesh,
    scratch_types=[
        pltpu.VMEM((_BPW * _T,), jnp.int32),
        pltpu.VMEM((_ROWS, _D), jnp.float32),
        pltpu.VMEM((_ROWS, _D), jnp.float32),
        pltpu.VMEM((_BPW, _D), jnp.float32),
        pltpu.VMEM((_BPW,), jnp.int32),
        pltpu.VMEM((_BPW,), jnp.int32),
        pltpu.VMEM((_BPW,), jnp.int32),
        pltpu.VMEM((_BPW, 64), jnp.float32),
        pltpu.VMEM((_BPW, 32), jnp.float32),
        pltpu.VMEM((_BPW, _D), jnp.float32),
        pltpu.VMEM((_BPW,), jnp.float32),
        pltpu.SemaphoreType.DMA,
        pltpu.SemaphoreType.DMA,
        pltpu.SemaphoreType.DMA,
    ],
    compiler_params=pltpu.CompilerParams(use_tc_tiling_on_sc=False),
)
def _sc_gather(seq_idx_hbm, user_id_hbm, cat_id_hbm, doc_id_hbm,
               user_tab, cat_tab, doc_tab, fbias,
               avg_out, user_out, cat_out, doc_out, bias_out,
               seqidx_v, rows0_v, rows1_v, avg_v,
               uidx_v, cidx_v, didx_v,
               urows_v, crows_v, drows_v, bvals_v,
               sem0, sem1, sem_s):
    wid = lax.axis_index("s") * _NC + lax.axis_index("c")
    base = wid * _BPW

    pltpu.sync_copy(seq_idx_hbm.at[pl.ds(base * _T, _BPW * _T)], seqidx_v)
    pltpu.sync_copy(user_id_hbm.at[pl.ds(base, _BPW)], uidx_v)
    pltpu.sync_copy(cat_id_hbm.at[pl.ds(base, _BPW)], cidx_v)
    pltpu.sync_copy(doc_id_hbm.at[pl.ds(base, _BPW)], didx_v)

    cp_u = pltpu.async_copy(user_tab.at[uidx_v], urows_v, sem_s)
    cp_c = pltpu.async_copy(cat_tab.at[cidx_v], crows_v, sem_s)
    cp_d = pltpu.async_copy(doc_tab.at[didx_v], drows_v, sem_s)
    cp_b = pltpu.async_copy(fbias.at[didx_v], bvals_v, sem_s)

    bufs = (rows0_v, rows1_v)
    sems = (sem0, sem1)
    cps = [None, None]
    cps[0] = pltpu.async_copy(doc_tab.at[seqidx_v.at[pl.ds(0, _ROWS)]],
                              bufs[0], sems[0])
    for ci in range(_NCHUNK):
        cur = ci % 2
        if ci + 1 < _NCHUNK:
            nxt = (ci + 1) % 2
            cps[nxt] = pltpu.async_copy(
                doc_tab.at[seqidx_v.at[pl.ds((ci + 1) * _ROWS, _ROWS)]],
                bufs[nxt], sems[nxt])
        cps[cur].wait()
        rows_v = bufs[cur]
        for j in range(_CB):
            def body(t, accs, _j=j, _rows=rows_v):
                r = _j * _T + t
                return (accs[0] + _rows[r, pl.ds(0, 16)],
                        accs[1] + _rows[r, pl.ds(16, 16)],
                        accs[2] + _rows[r, pl.ds(32, 16)],
                        accs[3] + _rows[r, pl.ds(48, 16)])
            z = jnp.zeros((16,), jnp.float32)
            a0, a1, a2, a3 = lax.fori_loop(0, _T, body, (z, z, z, z))
            o = ci * _CB + j
            scale = jnp.float32(1.0 / _T)
            avg_v[o, pl.ds(0, 16)] = a0 * scale
            avg_v[o, pl.ds(16, 16)] = a1 * scale
            avg_v[o, pl.ds(32, 16)] = a2 * scale
            avg_v[o, pl.ds(48, 16)] = a3 * scale

    pltpu.sync_copy(avg_v, avg_out.at[pl.ds(base, _BPW)])
    cp_u.wait()
    cp_c.wait()
    cp_d.wait()
    cp_b.wait()
    pltpu.sync_copy(urows_v, user_out.at[pl.ds(base, _BPW)])
    pltpu.sync_copy(crows_v, cat_out.at[pl.ds(base, _BPW)])
    pltpu.sync_copy(drows_v, doc_out.at[pl.ds(base, _BPW)])
    pltpu.sync_copy(bvals_v, bias_out.at[pl.ds(base, _BPW)])


def _tc_body(avg, user, cat, dense, doc, bias,
             w1, b1, w2, b2, w3, b3, out):
    f32 = jnp.float32
    h = jnp.dot(avg[...], w1[pl.ds(0, 64), :], preferred_element_type=f32)
    h += jnp.dot(user[...], w1[pl.ds(64, 64), :], preferred_element_type=f32)
    h += jnp.dot(cat[...], w1[pl.ds(128, 32), :], preferred_element_type=f32)
    h += jnp.dot(dense[...], w1[pl.ds(160, 16), :], preferred_element_type=f32)
    h = jnp.maximum(h + b1[...], 0.0)
    h = jnp.maximum(jnp.dot(h, w2[...], preferred_element_type=f32) + b2[...], 0.0)
    h = jnp.maximum(jnp.dot(h, w3[...], preferred_element_type=f32) + b3[...], 0.0)
    logit = jnp.sum(h * doc[...], axis=1, keepdims=True) + bias[...]
    out[...] = jax.nn.sigmoid(logit)


def kernel(user_id, item_cat, doc_id_seq, doc_id, dense_0,
           user_table, cat_table, doc_table, W1, b1, W2, b2, W3, b3,
           final_bias):
    seq_flat = doc_id_seq.reshape(-1).astype(jnp.int32)
    uid = user_id.astype(jnp.int32)
    cid = item_cat.astype(jnp.int32)
    did = doc_id.astype(jnp.int32)

    avg, uemb, cemb, demb, fb = _sc_gather(
        seq_flat, uid, cid, did, user_table, cat_table, doc_table, final_bias)

    out = pl.pallas_call(
        _tc_body,
        out_shape=jax.ShapeDtypeStruct((_B, 1), jnp.float32),
    )(avg, uemb, cemb, dense_0, demb, fb.reshape(_B, 1),
      W1, b1.reshape(1, -1), W2, b2.reshape(1, -1), W3, b3.reshape(1, -1))
    return out


# speedup vs baseline: 1.0003x; 1.0003x over previous
import functools

import jax
import jax.numpy as jnp
from jax import lax
from jax.experimental import pallas as pl
from jax.experimental.pallas import tpu as pltpu
from jax.experimental.pallas import tpu_sc as plsc

_B = 4096
_T = 50
_D = 64
_NC = 2
_NS = 16
_NW = _NC * _NS
_BPW = _B // _NW
_CB = 8
_NCHUNK = _BPW // _CB
_ROWS = _CB * _T

_mesh = plsc.VectorSubcoreMesh(core_axis_name="c", subcore_axis_name="s",
                               num_cores=_NC)


@functools.partial(
    pl.kernel,
    out_type=[
        jax.ShapeDtypeStruct((_B, _D), jnp.float32),
        jax.ShapeDtypeStruct((_B, 64), jnp.float32),
        jax.ShapeDtypeStruct((_B, 32), jnp.float32),
        jax.ShapeDtypeStruct((_B, _D), jnp.float32),
        jax.ShapeDtypeStruct((_B,), jnp.float32),
    ],
    mesh=_mesh,
    scratch_types=[
        pltpu.VMEM((_BPW, _T), jnp.int32),
        pltpu.VMEM((_T, _D), jnp.float32),
        pltpu.VMEM((_T, _D), jnp.float32),
        pltpu.VMEM((_T, _D), jnp.float32),
        pltpu.VMEM((_T, _D), jnp.float32),
        pltpu.VMEM((_BPW, _D), jnp.float32),
        pltpu.VMEM((_BPW,), jnp.int32),
        pltpu.VMEM((_BPW,), jnp.int32),
        pltpu.VMEM((_BPW,), jnp.int32),
        pltpu.VMEM((_BPW, 64), jnp.float32),
        pltpu.VMEM((_BPW, 32), jnp.float32),
        pltpu.VMEM((_BPW, _D), jnp.float32),
        pltpu.VMEM((_BPW,), jnp.float32),
        pltpu.SemaphoreType.DMA,
        pltpu.SemaphoreType.DMA,
        pltpu.SemaphoreType.DMA,
        pltpu.SemaphoreType.DMA,
        pltpu.SemaphoreType.DMA,
    ],
    compiler_params=pltpu.CompilerParams(use_tc_tiling_on_sc=False),
)
def _sc_gather(seq_idx_hbm, user_id_hbm, cat_id_hbm, doc_id_hbm,
               user_tab, cat_tab, doc_tab, fbias,
               avg_out, user_out, cat_out, doc_out, bias_out,
               seqidx_v, rows0_v, rows1_v, rows2_v, rows3_v, avg_v,
               uidx_v, cidx_v, didx_v,
               urows_v, crows_v, drows_v, bvals_v,
               sem0, sem1, sem2, sem3, sem_s):
    wid = lax.axis_index("s") * _NC + lax.axis_index("c")
    base = wid * _BPW

    pltpu.sync_copy(seq_idx_hbm.at[pl.ds(base, _BPW)], seqidx_v)
    pltpu.sync_copy(user_id_hbm.at[pl.ds(base, _BPW)], uidx_v)
    pltpu.sync_copy(cat_id_hbm.at[pl.ds(base, _BPW)], cidx_v)
    pltpu.sync_copy(doc_id_hbm.at[pl.ds(base, _BPW)], didx_v)

    cp_u = pltpu.async_copy(user_tab.at[uidx_v], urows_v, sem_s)
    cp_c = pltpu.async_copy(cat_tab.at[cidx_v], crows_v, sem_s)
    cp_d = pltpu.async_copy(doc_tab.at[didx_v], drows_v, sem_s)
    cp_b = pltpu.async_copy(fbias.at[didx_v], bvals_v, sem_s)

    nbuf = 4
    bufs = (rows0_v, rows1_v, rows2_v, rows3_v)
    sems = (sem0, sem1, sem2, sem3)
    for b in range(nbuf):
        pltpu.make_async_copy(doc_tab.at[seqidx_v.at[b]],
                              bufs[b], sems[b]).start()

    @pl.loop(0, _BPW // nbuf)
    def _chunk(ko):
        for b in range(nbuf):
            k = ko * nbuf + b
            pltpu.make_async_copy(doc_tab.at[seqidx_v.at[k]],
                                  bufs[b], sems[b]).wait()

            def body(t, accs, _rows=bufs[b]):
                return (accs[0] + _rows[t, pl.ds(0, 16)],
                        accs[1] + _rows[t, pl.ds(16, 16)],
                        accs[2] + _rows[t, pl.ds(32, 16)],
                        accs[3] + _rows[t, pl.ds(48, 16)])
            z = jnp.zeros((16,), jnp.float32)
            a0, a1, a2, a3 = lax.fori_loop(0, _T, body, (z, z, z, z))
            kn = k + nbuf

            @pl.when(kn < _BPW)
            def _refill(_b=b, _kn=kn):
                pltpu.make_async_copy(doc_tab.at[seqidx_v.at[_kn]],
                                      bufs[_b], sems[_b]).start()

            scale = jnp.float32(1.0 / _T)
            avg_v[k, pl.ds(0, 16)] = a0 * scale
            avg_v[k, pl.ds(16, 16)] = a1 * scale
            avg_v[k, pl.ds(32, 16)] = a2 * scale
            avg_v[k, pl.ds(48, 16)] = a3 * scale

    pltpu.sync_copy(avg_v, avg_out.at[pl.ds(base, _BPW)])
    cp_u.wait()
    cp_c.wait()
    cp_d.wait()
    cp_b.wait()
    pltpu.sync_copy(urows_v, user_out.at[pl.ds(base, _BPW)])
    pltpu.sync_copy(crows_v, cat_out.at[pl.ds(base, _BPW)])
    pltpu.sync_copy(drows_v, doc_out.at[pl.ds(base, _BPW)])
    pltpu.sync_copy(bvals_v, bias_out.at[pl.ds(base, _BPW)])


def _tc_body(avg, user, cat, dense, doc, bias,
             w1, b1, w2, b2, w3, b3, out):
    f32 = jnp.float32
    h = jnp.dot(avg[...], w1[pl.ds(0, 64), :], preferred_element_type=f32)
    h += jnp.dot(user[...], w1[pl.ds(64, 64), :], preferred_element_type=f32)
    h += jnp.dot(cat[...], w1[pl.ds(128, 32), :], preferred_element_type=f32)
    h += jnp.dot(dense[...], w1[pl.ds(160, 16), :], preferred_element_type=f32)
    h = jnp.maximum(h + b1[...].reshape(1, -1), 0.0)
    h = jnp.maximum(jnp.dot(h, w2[...], preferred_element_type=f32)
                    + b2[...].reshape(1, -1), 0.0)
    h = jnp.maximum(jnp.dot(h, w3[...], preferred_element_type=f32)
                    + b3[...].reshape(1, -1), 0.0)
    logit = jnp.sum(h * doc[...], axis=1, keepdims=True) + bias[...]
    out[...] = jax.nn.sigmoid(logit)


def kernel(user_id, item_cat, doc_id_seq, doc_id, dense_0,
           user_table, cat_table, doc_table, W1, b1, W2, b2, W3, b3,
           final_bias):
    uid = user_id.astype(jnp.int32)
    cid = item_cat.astype(jnp.int32)
    did = doc_id.astype(jnp.int32)

    avg, uemb, cemb, demb, fb = _sc_gather(
        doc_id_seq.astype(jnp.int32), uid, cid, did,
        user_table, cat_table, doc_table, final_bias)

    out = pl.pallas_call(
        _tc_body,
        out_shape=jax.ShapeDtypeStruct((_B, 1), jnp.float32),
    )(avg, uemb, cemb, dense_0, demb, fb.reshape(_B, 1),
      W1, b1, W2, b2, W3, b3)
    return out
